# Initial kernel scaffold; baseline (speedup 1.0000x reference)
import jax
import jax.numpy as jnp
from jax.experimental import pallas as pl


def kernel(rpn_bbox_deltas, rpn_labels):
    raise NotImplementedError("write your pallas kernel here")


# speedup vs baseline: 75.3179x; 75.3179x over previous
import functools

import numpy as np
import jax
import jax.numpy as jnp
from jax.experimental import pallas as pl

PRE_NMS_TOPN = 6000
POST_NMS_TOPN = 300
IOU_THRESHOLD = 0.7
_VARIANCES = np.array([0.1, 0.1, 0.2, 0.2], dtype=np.float32)

_FH = _FW = 47
_NUM_ANCH = 9
_TOTAL = _FH * _FW * _NUM_ANCH
_ROWS = 156
_LANES = 128
_NPAD = _ROWS * _LANES
_OUT_PAD = 304


def _make_anchor_consts():
    scales = [0.125, 0.25, 0.5]
    ratios = [0.5, 1.0, 2.0]
    out = []
    for s in scales:
        for r in ratios:
            w = s * np.sqrt(r)
            h = s / np.sqrt(r)
            out.append([-h / 2.0, -w / 2.0, h / 2.0, w / 2.0])
    base = np.array(out, dtype=np.float32)

    cy = ((np.arange(_FH, dtype=np.float32) + np.float32(0.5))
          / np.float32(_FH))
    cx = ((np.arange(_FW, dtype=np.float32) + np.float32(0.5))
          / np.float32(_FW))
    gx, gy = np.meshgrid(cx, cy)
    fy = gy.reshape(-1)
    fx = gx.reshape(-1)
    grid = np.stack([fy, fx, fy, fx], axis=-1)
    anchors = base.reshape(1, -1, 4) + grid.reshape(-1, 1, 4)
    anchors = np.clip(anchors.reshape(-1, 4), np.float32(0.0),
                      np.float32(1.0)).astype(np.float32)

    aw = anchors[:, 3] - anchors[:, 1]
    ah = anchors[:, 2] - anchors[:, 0]
    acx = anchors[:, 1] + np.float32(0.5) * aw
    acy = anchors[:, 0] + np.float32(0.5) * ah

    def padrc(v):
        p = np.zeros((_NPAD,), dtype=np.float32)
        p[:_TOTAL] = v
        return p.reshape(_ROWS, _LANES)

    return np.stack([padrc(aw), padrc(ah), padrc(acx), padrc(acy)], axis=0)


_ANCHOR_CONSTS = _make_anchor_consts()


def _nms_kernel(deltas_ref, labels_ref, anch_ref, out_ref):
    d0 = deltas_ref[0, 0] * _VARIANCES[0]
    d1 = deltas_ref[0, 1] * _VARIANCES[1]
    d2 = deltas_ref[0, 2] * _VARIANCES[2]
    d3 = deltas_ref[0, 3] * _VARIANCES[3]
    aw = anch_ref[0]
    ah = anch_ref[1]
    acx = anch_ref[2]
    acy = anch_ref[3]

    bw = jnp.exp(d3) * aw
    bh = jnp.exp(d2) * ah
    bcx = d1 * aw + acx
    bcy = d0 * ah + acy
    y1 = bcy - 0.5 * bh
    x1 = bcx - 0.5 * bw
    y2 = bh + y1
    x2 = bw + x1
    area = (y2 - y1) * (x2 - x1)

    s = labels_ref[0]
    flat_idx = (
        jax.lax.broadcasted_iota(jnp.int32, (_ROWS, _LANES), 0) * _LANES
        + jax.lax.broadcasted_iota(jnp.int32, (_ROWS, _LANES), 1))

    def bs_body(_, state):
        lo, hi = state
        mid = (lo + hi) // 2
        tv = jax.lax.bitcast_convert_type(jnp.reshape(mid, (1, 1)),
                                          jnp.float32)
        cnt = jnp.sum((s >= tv).astype(jnp.int32))
        big = cnt >= PRE_NMS_TOPN
        return (jnp.where(big, mid, lo), jnp.where(big, hi, mid))

    lo0 = jnp.int32(0)
    hi0 = jnp.int32(0x3F800000)
    lo, hi = jax.lax.fori_loop(0, 30, bs_body, (lo0, hi0))
    thr = jax.lax.bitcast_convert_type(jnp.reshape(lo, (1, 1)), jnp.float32)

    eq = s == thr
    c_gt = jnp.sum((s > thr).astype(jnp.int32))
    need = PRE_NMS_TOPN - c_gt

    def ibs_body(_, state):
        lo_i, hi_i = state
        mid = (lo_i + hi_i) // 2
        cnt = jnp.sum((eq & (flat_idx <= mid)).astype(jnp.int32))
        ok = cnt >= need
        return (jnp.where(ok, lo_i, mid), jnp.where(ok, mid, hi_i))

    lo_i, hi_i = jax.lax.fori_loop(
        0, 16, ibs_body, (jnp.int32(-1), jnp.int32(_NPAD - 1)))
    in_set = (s > thr) | (eq & (flat_idx <= hi_i))

    neg_inf = jnp.float32(-jnp.inf)
    s_cur0 = jnp.where(in_set, s, neg_inf)

    rows8 = jax.lax.broadcasted_iota(jnp.int32, (8, 1), 0)
    r0 = (rows8 == 0).astype(jnp.float32)
    r1 = (rows8 == 1).astype(jnp.float32)
    r2 = (rows8 == 2).astype(jnp.float32)
    r3 = (rows8 == 3).astype(jnp.float32)
    col_iota = jax.lax.broadcasted_iota(jnp.int32, (1, _OUT_PAD), 1)

    def body(k, carry):
        s_cur, out = carry
        m = jnp.max(s_cur)
        valid = m > neg_inf
        i = jnp.min(jnp.where(s_cur == m, flat_idx, jnp.int32(_NPAD)))
        onesel = (flat_idx == i).astype(jnp.float32)
        by1 = jnp.sum(y1 * onesel)
        bx1 = jnp.sum(x1 * onesel)
        by2 = jnp.sum(y2 * onesel)
        bx2 = jnp.sum(x2 * onesel)
        barea = jnp.sum(area * onesel)

        yy1 = jnp.maximum(by1, y1)
        xx1 = jnp.maximum(bx1, x1)
        yy2 = jnp.minimum(by2, y2)
        xx2 = jnp.minimum(bx2, x2)
        inter = (jnp.maximum(yy2 - yy1, 0.0) * jnp.maximum(xx2 - xx1, 0.0))
        union = barea + area - inter
        iou = inter / jnp.maximum(union, 1e-8)
        sup = (iou > IOU_THRESHOLD) & valid
        s_new = jnp.where(sup, neg_inf, s_cur)

        vf = jnp.where(valid, 1.0, 0.0)
        cy1 = jnp.clip(by1, 0.0, 1.0) * vf
        cx1 = jnp.clip(bx1, 0.0, 1.0) * vf
        cy2 = jnp.clip(by2, 0.0, 1.0) * vf
        cx2 = jnp.clip(bx2, 0.0, 1.0) * vf
        valcol = cy1 * r0 + cx1 * r1 + cy2 * r2 + cx2 * r3
        oh = (col_iota == k).astype(jnp.float32)
        out = out + valcol * oh
        return (s_new, out)

    out0 = jnp.zeros((8, _OUT_PAD), dtype=jnp.float32)
    _, out = jax.lax.fori_loop(0, POST_NMS_TOPN, body, (s_cur0, out0))
    out_ref[0] = out


@functools.partial(jax.jit, static_argnames=())
def kernel(rpn_bbox_deltas, rpn_labels):
    b = rpn_bbox_deltas.shape[0]
    deltas = rpn_bbox_deltas.reshape(b, _TOTAL, 4).transpose(0, 2, 1)
    deltas = jnp.pad(deltas, ((0, 0), (0, 0), (0, _NPAD - _TOTAL)))
    deltas = deltas.reshape(b, 4, _ROWS, _LANES)
    labels = rpn_labels.reshape(b, _TOTAL)
    labels = jnp.pad(labels, ((0, 0), (0, _NPAD - _TOTAL)),
                     constant_values=-1.0)
    labels = labels.reshape(b, _ROWS, _LANES)
    anch = jnp.asarray(_ANCHOR_CONSTS)

    out = pl.pallas_call(
        _nms_kernel,
        grid=(b,),
        in_specs=[
            pl.BlockSpec((1, 4, _ROWS, _LANES), lambda i: (i, 0, 0, 0)),
            pl.BlockSpec((1, _ROWS, _LANES), lambda i: (i, 0, 0)),
            pl.BlockSpec((4, _ROWS, _LANES), lambda i: (0, 0, 0)),
        ],
        out_specs=pl.BlockSpec((1, 8, _OUT_PAD), lambda i: (i, 0, 0)),
        out_shape=jax.ShapeDtypeStruct((b, 8, _OUT_PAD), jnp.float32),
    )(deltas, labels, anch)

    roi = out[:, :4, :POST_NMS_TOPN].transpose(0, 2, 1)
    return jax.lax.stop_gradient(roi)


# speedup vs baseline: 183.5563x; 2.4371x over previous
import numpy as np
import jax
import jax.numpy as jnp
from jax.experimental import pallas as pl

PRE_NMS_TOPN = 6000
POST_NMS_TOPN = 300
IOU_THRESHOLD = 0.7
_VARIANCES = np.array([0.1, 0.1, 0.2, 0.2], dtype=np.float32)

_FH = _FW = 47
_NUM_ANCH = 9
_TOTAL = _FH * _FW * _NUM_ANCH
_ROWS = 156
_LANES = 128
_NPAD = _ROWS * _LANES
_OUT_PAD = 304
_B = 4


def _make_anchor_consts():
    scales = [0.125, 0.25, 0.5]
    ratios = [0.5, 1.0, 2.0]
    out = []
    for s in scales:
        for r in ratios:
            w = s * np.sqrt(r)
            h = s / np.sqrt(r)
            out.append([-h / 2.0, -w / 2.0, h / 2.0, w / 2.0])
    base = np.array(out, dtype=np.float32)

    cy = ((np.arange(_FH, dtype=np.float32) + np.float32(0.5))
          / np.float32(_FH))
    cx = ((np.arange(_FW, dtype=np.float32) + np.float32(0.5))
          / np.float32(_FW))
    gx, gy = np.meshgrid(cx, cy)
    fy = gy.reshape(-1)
    fx = gx.reshape(-1)
    grid = np.stack([fy, fx, fy, fx], axis=-1)
    anchors = base.reshape(1, -1, 4) + grid.reshape(-1, 1, 4)
    anchors = np.clip(anchors.reshape(-1, 4), np.float32(0.0),
                      np.float32(1.0)).astype(np.float32)

    aw = anchors[:, 3] - anchors[:, 1]
    ah = anchors[:, 2] - anchors[:, 0]
    acx = anchors[:, 1] + np.float32(0.5) * aw
    acy = anchors[:, 0] + np.float32(0.5) * ah

    def padrc(v):
        p = np.zeros((_NPAD,), dtype=np.float32)
        p[:_TOTAL] = v
        return p.reshape(_ROWS, _LANES)

    return np.stack([padrc(aw), padrc(ah), padrc(acx), padrc(acy)], axis=0)


_ANCHOR_CONSTS = _make_anchor_consts()


def _nms_kernel(deltas_ref, labels_ref, anch_ref, out_ref):
    d0 = deltas_ref[:, 0] * _VARIANCES[0]
    d1 = deltas_ref[:, 1] * _VARIANCES[1]
    d2 = deltas_ref[:, 2] * _VARIANCES[2]
    d3 = deltas_ref[:, 3] * _VARIANCES[3]
    aw = anch_ref[0]
    ah = anch_ref[1]
    acx = anch_ref[2]
    acy = anch_ref[3]

    bw = jnp.exp(d3) * aw
    bh = jnp.exp(d2) * ah
    bcx = d1 * aw + acx
    bcy = d0 * ah + acy
    y1 = bcy - 0.5 * bh
    x1 = bcx - 0.5 * bw
    y2 = bh + y1
    x2 = bw + x1
    area = (y2 - y1) * (x2 - x1)

    s = labels_ref[...]
    flat_idx = (
        jax.lax.broadcasted_iota(jnp.int32, (1, _ROWS, _LANES), 1) * _LANES
        + jax.lax.broadcasted_iota(jnp.int32, (1, _ROWS, _LANES), 2))

    def psum(x):
        return jnp.sum(x, axis=(1, 2), keepdims=True)

    def bs_body(_, state):
        lo, hi = state
        mid = (lo + hi) // 2
        tv = jax.lax.bitcast_convert_type(mid, jnp.float32)
        big = psum((s >= tv).astype(jnp.int32)) >= PRE_NMS_TOPN
        return (jnp.where(big, mid, lo), jnp.where(big, hi, mid))

    lo0 = jnp.full((_B, 1, 1), 0, jnp.int32)
    hi0 = jnp.full((_B, 1, 1), 0x3F800000, jnp.int32)
    lo, hi = jax.lax.fori_loop(0, 30, bs_body, (lo0, hi0))
    thr = jax.lax.bitcast_convert_type(lo, jnp.float32)

    eq = s == thr
    need = PRE_NMS_TOPN - psum((s > thr).astype(jnp.int32))

    def ibs_body(_, state):
        lo_i, hi_i = state
        mid = (lo_i + hi_i) // 2
        cnt = psum((eq & (flat_idx <= mid)).astype(jnp.int32))
        ok = cnt >= need
        return (jnp.where(ok, lo_i, mid), jnp.where(ok, mid, hi_i))

    lo_i, hi_i = jax.lax.fori_loop(
        0, 16, ibs_body,
        (jnp.full((_B, 1, 1), -1, jnp.int32),
         jnp.full((_B, 1, 1), _NPAD - 1, jnp.int32)))
    in_set = (s > thr) | (eq & (flat_idx <= hi_i))

    neg_inf = jnp.float32(-jnp.inf)
    s_cur0 = jnp.where(in_set, s, neg_inf)

    rows8 = jax.lax.broadcasted_iota(jnp.int32, (8, 1), 0)
    r0 = (rows8 == 0).astype(jnp.float32)
    r1 = (rows8 == 1).astype(jnp.float32)
    r2 = (rows8 == 2).astype(jnp.float32)
    r3 = (rows8 == 3).astype(jnp.float32)
    col_iota = jax.lax.broadcasted_iota(jnp.int32, (1, _OUT_PAD), 1)

    def body(k, carry):
        s_cur, out = carry
        m = jnp.max(s_cur, axis=(1, 2), keepdims=True)
        valid = m > neg_inf
        i = jnp.min(jnp.where(s_cur == m, flat_idx, jnp.int32(_NPAD)),
                    axis=(1, 2), keepdims=True)
        onesel = (flat_idx == i).astype(jnp.float32)
        by1 = psum(y1 * onesel)
        bx1 = psum(x1 * onesel)
        by2 = psum(y2 * onesel)
        bx2 = psum(x2 * onesel)
        barea = (by2 - by1) * (bx2 - bx1)

        yy1 = jnp.maximum(by1, y1)
        xx1 = jnp.maximum(bx1, x1)
        yy2 = jnp.minimum(by2, y2)
        xx2 = jnp.minimum(bx2, x2)
        inter = (jnp.maximum(yy2 - yy1, 0.0) * jnp.maximum(xx2 - xx1, 0.0))
        union = barea + area - inter
        iou = inter / jnp.maximum(union, 1e-8)
        sup = (iou > IOU_THRESHOLD) & valid
        s_new = jnp.where(sup, neg_inf, s_cur)

        vf = valid.astype(jnp.float32)
        cy1 = jnp.clip(by1, 0.0, 1.0) * vf
        cx1 = jnp.clip(bx1, 0.0, 1.0) * vf
        cy2 = jnp.clip(by2, 0.0, 1.0) * vf
        cx2 = jnp.clip(bx2, 0.0, 1.0) * vf
        valcol = cy1 * r0 + cx1 * r1 + cy2 * r2 + cx2 * r3
        oh = (col_iota == k).astype(jnp.float32)
        out = out + valcol * oh
        return (s_new, out)

    out0 = jnp.zeros((_B, 8, _OUT_PAD), dtype=jnp.float32)
    _, out = jax.lax.fori_loop(0, POST_NMS_TOPN, body, (s_cur0, out0))
    out_ref[...] = out


@jax.jit
def kernel(rpn_bbox_deltas, rpn_labels):
    b = rpn_bbox_deltas.shape[0]
    deltas = rpn_bbox_deltas.reshape(b, _TOTAL, 4).transpose(0, 2, 1)
    deltas = jnp.pad(deltas, ((0, 0), (0, 0), (0, _NPAD - _TOTAL)))
    deltas = deltas.reshape(b, 4, _ROWS, _LANES)
    labels = rpn_labels.reshape(b, _TOTAL)
    labels = jnp.pad(labels, ((0, 0), (0, _NPAD - _TOTAL)),
                     constant_values=-1.0)
    labels = labels.reshape(b, _ROWS, _LANES)
    anch = jnp.asarray(_ANCHOR_CONSTS)

    out = pl.pallas_call(
        _nms_kernel,
        out_shape=jax.ShapeDtypeStruct((b, 8, _OUT_PAD), jnp.float32),
    )(deltas, labels, anch)

    roi = out[:, :4, :POST_NMS_TOPN].transpose(0, 2, 1)
    return jax.lax.stop_gradient(roi)
